# baseline (speedup 1.0000x reference)
import functools

import numpy as np
import jax
import jax.numpy as jnp
from jax import lax
from jax.experimental import pallas as pl
from jax.experimental.pallas import tpu as pltpu
from jax.experimental.pallas import tpu_sc as plsc

_D = 64
_NC, _NS = 2, 16
_NW = _NC * _NS
_EPS = 1e-12


@functools.lru_cache(maxsize=None)
def _pe_const(seq_len: int):
    position = np.arange(0, seq_len, dtype=np.float32)[:, None]
    div_term = np.exp(np.arange(0, _D, 2, dtype=np.float32) * -(np.log(10000.0) / _D))
    pe = np.zeros((seq_len, _D), dtype=np.float32)
    pe[:, 0::2] = np.sin(position * div_term)
    pe[:, 1::2] = np.cos(position * div_term)
    return pe[None, :, :]


@functools.lru_cache(maxsize=None)
def _gather_call(n_rows: int, chunk: int):
    assert n_rows % (_NW * chunk) == 0 and chunk % 8 == 0
    n_chunks = n_rows // (_NW * chunk)
    rows_per_w = n_chunks * chunk
    mesh = plsc.VectorSubcoreMesh(
        core_axis_name="c", subcore_axis_name="s",
        num_cores=_NC, num_subcores=_NS)

    @functools.partial(
        pl.kernel,
        out_type=jax.ShapeDtypeStruct((n_rows, _D), jnp.float32),
        mesh=mesh,
        scratch_types=[
            pltpu.VMEM((chunk,), jnp.int32),
            pltpu.VMEM((chunk, _D), jnp.float32),
            pltpu.SemaphoreType.DMA,
        ],
        compiler_params=pltpu.CompilerParams(use_tc_tiling_on_sc=False),
    )
    def k(idx_hbm, table_hbm, out_hbm, idx_v, rows_v, sem):
        wid = lax.axis_index("s") * _NC + lax.axis_index("c")
        base0 = wid * rows_per_w

        def body(i, carry):
            base = base0 + i * chunk
            pltpu.sync_copy(idx_hbm.at[pl.ds(base, chunk)], idx_v)
            pltpu.async_copy(table_hbm.at[idx_v], rows_v, sem).wait()
            pltpu.sync_copy(rows_v, out_hbm.at[pl.ds(base, chunk)])
            return carry

        lax.fori_loop(0, n_chunks, body, 0, unroll=False)

    return k


def _ln_body(emb_ref, pe_ref, g_ref, b_ref, out_ref):
    x = emb_ref[...] + pe_ref[...]
    mu = jnp.mean(x, axis=-1, keepdims=True)
    xc = x - mu
    var = jnp.mean(xc * xc, axis=-1, keepdims=True)
    out_ref[...] = xc * lax.rsqrt(var + _EPS) * g_ref[...] + b_ref[...]


@functools.lru_cache(maxsize=None)
def _ln_call(batch: int, seq: int, block_b: int):
    grid = (batch // block_b,)
    return pl.pallas_call(
        _ln_body,
        grid=grid,
        in_specs=[
            pl.BlockSpec((block_b, seq, _D), lambda i: (i, 0, 0)),
            pl.BlockSpec((1, seq, _D), lambda i: (0, 0, 0)),
            pl.BlockSpec((1, 1, _D), lambda i: (0, 0, 0)),
            pl.BlockSpec((1, 1, _D), lambda i: (0, 0, 0)),
        ],
        out_specs=pl.BlockSpec((block_b, seq, _D), lambda i: (i, 0, 0)),
        out_shape=jax.ShapeDtypeStruct((batch, seq, _D), jnp.float32),
    )


def kernel(input_ids, table, gamma, beta):
    B, S = input_ids.shape
    ids_flat = input_ids.reshape(-1).astype(jnp.int32)
    emb = _gather_call(B * S, 800)(ids_flat, table)
    emb = emb.reshape(B, S, _D)
    pe = jnp.asarray(_pe_const(S))
    out = _ln_call(B, S, 32)(
        emb, pe, gamma.reshape(1, 1, _D), beta.reshape(1, 1, _D))
    return out
